# baseline (speedup 1.0000x reference)
import functools

import jax
import jax.numpy as jnp
from jax import lax
from jax.experimental import pallas as pl
from jax.experimental.pallas import tpu as pltpu
from jax.experimental.pallas import tpu_sc as plsc

N = 10000
E = 320000
D = 128

NC = 2
NS = 16
NW = NC * NS
EPW = E // NW
CHUNK = 128
NFULL = EPW // CHUNK
TAIL = EPW - NFULL * CHUNK
BLK = 200
NBLK = N // BLK
DEGW = 16

_mesh = plsc.VectorSubcoreMesh(core_axis_name="c", subcore_axis_name="s")


def _blocked_copy(sid, make_src, make_dst):
    @pl.loop(0, NBLK)
    def _(j):
        @pl.when(j % NS == sid)
        def _():
            pltpu.sync_copy(make_src(pl.ds(j * BLK, BLK)),
                            make_dst(pl.ds(j * BLK, BLK)))


@functools.partial(
    pl.kernel,
    out_type=jax.ShapeDtypeStruct((NC, N, DEGW), jnp.float32),
    mesh=_mesh,
    scratch_types=[
        pltpu.VMEM((CHUNK,), jnp.int32),
        pltpu.VMEM((TAIL,), jnp.int32),
        pltpu.VMEM((CHUNK, DEGW), jnp.float32),
        pltpu.VMEM((BLK, DEGW), jnp.float32),
        pltpu.VMEM_SHARED((N, DEGW), jnp.float32),
    ],
)
def _degree_kernel(dst_hbm, out_hbm, idx_v, idxt_v, ones_v, z_v, acc):
    cid = lax.axis_index("c")
    sid = lax.axis_index("s")

    one16 = jnp.full((16,), 1.0, jnp.float32)
    zero16 = jnp.zeros((16,), jnp.float32)

    @pl.loop(0, CHUNK)
    def _(i):
        ones_v[i, :] = one16

    @pl.loop(0, BLK)
    def _(i):
        z_v[i, :] = zero16

    @pl.loop(0, NBLK)
    def _(j):
        @pl.when(j % NS == sid)
        def _():
            pltpu.sync_copy(z_v, acc.at[pl.ds(j * BLK, BLK)])
    plsc.subcore_barrier()

    base = (cid * NS + sid) * EPW

    @pl.loop(0, NFULL)
    def _(i):
        pltpu.sync_copy(dst_hbm.at[pl.ds(base + i * CHUNK, CHUNK)], idx_v)
        pltpu.sync_copy(ones_v, acc.at[idx_v], add=True)

    pltpu.sync_copy(dst_hbm.at[pl.ds(base + NFULL * CHUNK, TAIL)], idxt_v)
    pltpu.sync_copy(ones_v.at[pl.ds(0, TAIL)], acc.at[idxt_v], add=True)

    plsc.subcore_barrier()
    _blocked_copy(sid, lambda s: acc.at[s], lambda s: out_hbm.at[cid, s])


NB = 3
NGRP = NFULL // NB
NCHT = E // CHUNK


@functools.partial(
    pl.kernel,
    out_type=jax.ShapeDtypeStruct((NC, N, D), jnp.float32),
    mesh=_mesh,
    scratch_types=[
        pltpu.VMEM((NB, 2, CHUNK), jnp.int32),
        pltpu.VMEM((2, CHUNK), jnp.int32),
        [pltpu.VMEM((CHUNK, D), jnp.float32)] * NB,
        [pltpu.SemaphoreType.DMA] * NB,
        [pltpu.SemaphoreType.DMA] * NB,
        pltpu.VMEM_SHARED((N, D), jnp.float32),
    ],
)
def _aggregate_kernel(table_hbm, sdg_hbm, sdt_hbm, out_hbm,
                      gsd, sdi, rows, gsem, ssem, acc):
    cid = lax.axis_index("c")
    sid = lax.axis_index("s")
    wid = cid * NS + sid

    _blocked_copy(sid, lambda s: table_hbm.at[s], lambda s: acc.at[s])
    plsc.subcore_barrier()

    gbase = wid * NGRP

    @pl.loop(0, NGRP)
    def _(g):
        pltpu.sync_copy(sdg_hbm.at[gbase + g], gsd)
        for b in range(NB):
            pltpu.async_copy(table_hbm.at[gsd.at[b, 0]], rows[b], gsem[b])
        for b in range(NB):
            pltpu.make_async_copy(table_hbm.at[gsd.at[b, 0]], rows[b],
                                  gsem[b]).wait()
            pltpu.async_copy(rows[b], acc.at[gsd.at[b, 1]], ssem[b],
                             add=True)
        for b in range(NB):
            pltpu.make_async_copy(rows[b], acc.at[gsd.at[b, 1]],
                                  ssem[b]).wait()

    @pl.when(wid < NCHT - NW * NFULL)
    def _():
        pltpu.sync_copy(sdt_hbm.at[wid], sdi)
        pltpu.async_copy(table_hbm.at[sdi.at[0]], rows[0], gsem[0]).wait()
        pltpu.sync_copy(rows[0], acc.at[sdi.at[1]], add=True)

    plsc.subcore_barrier()
    _blocked_copy(sid, lambda s: acc.at[s], lambda s: out_hbm.at[cid, s])


_ROWS = 1000


def _mm_body(x_ref, w_ref, o_ref):
    o_ref[...] = lax.dot_general(
        x_ref[...], w_ref[...], (((1,), (0,)), ((), ())),
        preferred_element_type=jnp.float32,
        precision=lax.Precision.HIGHEST,
    )


def _matmul(x, w):
    return pl.pallas_call(
        _mm_body,
        grid=(N // _ROWS,),
        in_specs=[
            pl.BlockSpec((_ROWS, D), lambda i: (i, 0)),
            pl.BlockSpec((D, D), lambda i: (0, 0)),
        ],
        out_specs=pl.BlockSpec((_ROWS, D), lambda i: (i, 0)),
        out_shape=jax.ShapeDtypeStruct((N, D), jnp.float32),
    )(x, w)


def _dis_block(dp_ref):
    deg = 1.0 + dp_ref[0, :, 0:1] + dp_ref[1, :, 0:1]
    return lax.rsqrt(deg)


def _scale_body(dp_ref, h_ref, o_ref):
    o_ref[...] = h_ref[...] * _dis_block(dp_ref)


def _scale(deg_parts, h):
    return pl.pallas_call(
        _scale_body,
        grid=(N // _ROWS,),
        in_specs=[
            pl.BlockSpec((NC, _ROWS, DEGW), lambda i: (0, i, 0)),
            pl.BlockSpec((_ROWS, D), lambda i: (i, 0)),
        ],
        out_specs=pl.BlockSpec((_ROWS, D), lambda i: (i, 0)),
        out_shape=jax.ShapeDtypeStruct((N, D), jnp.float32),
    )(deg_parts, h)


def _mid_body(p_ref, h_ref, dp_ref, b1_ref, w2_ref, o_ref):
    dis = _dis_block(dp_ref)
    s = p_ref[0] + p_ref[1] - h_ref[...]
    z = jnp.maximum(s * dis + b1_ref[...], 0.0)
    o_ref[...] = lax.dot_general(
        z, w2_ref[...], (((1,), (0,)), ((), ())),
        preferred_element_type=jnp.float32,
        precision=lax.Precision.HIGHEST,
    ) * dis


def _mid(parts, h, deg_parts, b1, w2):
    return pl.pallas_call(
        _mid_body,
        grid=(N // _ROWS,),
        in_specs=[
            pl.BlockSpec((NC, _ROWS, D), lambda i: (0, i, 0)),
            pl.BlockSpec((_ROWS, D), lambda i: (i, 0)),
            pl.BlockSpec((NC, _ROWS, DEGW), lambda i: (0, i, 0)),
            pl.BlockSpec((1, D), lambda i: (0, 0)),
            pl.BlockSpec((D, D), lambda i: (0, 0)),
        ],
        out_specs=pl.BlockSpec((_ROWS, D), lambda i: (i, 0)),
        out_shape=jax.ShapeDtypeStruct((N, D), jnp.float32),
    )(parts, h, deg_parts, b1, w2)


def _final_body(p_ref, h_ref, dp_ref, b2_ref, o_ref):
    dis = _dis_block(dp_ref)
    s = p_ref[0] + p_ref[1] - h_ref[...]
    o_ref[...] = s * dis + b2_ref[...]


def _final(parts, h, deg_parts, b2):
    return pl.pallas_call(
        _final_body,
        grid=(N // _ROWS,),
        in_specs=[
            pl.BlockSpec((NC, _ROWS, D), lambda i: (0, i, 0)),
            pl.BlockSpec((_ROWS, D), lambda i: (i, 0)),
            pl.BlockSpec((NC, _ROWS, DEGW), lambda i: (0, i, 0)),
            pl.BlockSpec((1, D), lambda i: (0, 0)),
        ],
        out_specs=pl.BlockSpec((_ROWS, D), lambda i: (i, 0)),
        out_shape=jax.ShapeDtypeStruct((N, D), jnp.float32),
    )(parts, h, deg_parts, b2)


def kernel(x, edge_index, W1, b1, W2, b2):
    src = edge_index[0].astype(jnp.int32)
    dst = edge_index[1].astype(jnp.int32)
    b1r = b1.reshape(1, D)
    b2r = b2.reshape(1, D)

    sd = jnp.stack([src, dst]).reshape(2, NCHT, CHUNK).transpose(1, 0, 2)
    sdg = sd[:NW * NFULL].reshape(NW * NGRP, NB, 2, CHUNK)
    sdt = sd[NW * NFULL:]

    deg_parts = _degree_kernel(dst)
    h1_raw = _matmul(x, W1)
    h1 = _scale(deg_parts, h1_raw)
    p1 = _aggregate_kernel(h1, sdg, sdt)
    h2 = _mid(p1, h1, deg_parts, b1r, W2)
    p2 = _aggregate_kernel(h2, sdg, sdt)
    return _final(p2, h2, deg_parts, b2r)


# speedup vs baseline: 1.1244x; 1.1244x over previous
import functools

import jax
import jax.numpy as jnp
from jax import lax
from jax.experimental import pallas as pl
from jax.experimental.pallas import tpu as pltpu
from jax.experimental.pallas import tpu_sc as plsc

N = 10000
E = 320000
D = 128

NC = 2
NS = 16
NW = NC * NS
EPW = E // NW
CHUNK = 128
NFULL = EPW // CHUNK
TAIL = EPW - NFULL * CHUNK
BLK = 200
NBLK = N // BLK
DEGW = 16

_mesh = plsc.VectorSubcoreMesh(core_axis_name="c", subcore_axis_name="s")


def _blocked_copy(sid, make_src, make_dst):
    @pl.loop(0, NBLK)
    def _(j):
        @pl.when(j % NS == sid)
        def _():
            pltpu.sync_copy(make_src(pl.ds(j * BLK, BLK)),
                            make_dst(pl.ds(j * BLK, BLK)))


@functools.partial(
    pl.kernel,
    out_type=jax.ShapeDtypeStruct((NC, N, DEGW), jnp.float32),
    mesh=_mesh,
    scratch_types=[
        pltpu.VMEM((CHUNK,), jnp.int32),
        pltpu.VMEM((TAIL,), jnp.int32),
        pltpu.VMEM((CHUNK, DEGW), jnp.float32),
        pltpu.VMEM((BLK, DEGW), jnp.float32),
        pltpu.VMEM_SHARED((N, DEGW), jnp.float32),
    ],
)
def _degree_kernel(dst_hbm, out_hbm, idx_v, idxt_v, ones_v, z_v, acc):
    cid = lax.axis_index("c")
    sid = lax.axis_index("s")

    one16 = jnp.full((16,), 1.0, jnp.float32)
    zero16 = jnp.zeros((16,), jnp.float32)

    @pl.loop(0, CHUNK)
    def _(i):
        ones_v[i, :] = one16

    @pl.loop(0, BLK)
    def _(i):
        z_v[i, :] = zero16

    @pl.loop(0, NBLK)
    def _(j):
        @pl.when(j % NS == sid)
        def _():
            pltpu.sync_copy(z_v, acc.at[pl.ds(j * BLK, BLK)])
    plsc.subcore_barrier()

    base = (cid * NS + sid) * EPW

    @pl.loop(0, NFULL)
    def _(i):
        pltpu.sync_copy(dst_hbm.at[pl.ds(base + i * CHUNK, CHUNK)], idx_v)
        pltpu.sync_copy(ones_v, acc.at[idx_v], add=True)

    pltpu.sync_copy(dst_hbm.at[pl.ds(base + NFULL * CHUNK, TAIL)], idxt_v)
    pltpu.sync_copy(ones_v.at[pl.ds(0, TAIL)], acc.at[idxt_v], add=True)

    plsc.subcore_barrier()
    _blocked_copy(sid, lambda s: acc.at[s], lambda s: out_hbm.at[cid, s])


NB = 3
NGRP = NFULL // NB
NCHT = E // CHUNK


@functools.partial(
    pl.kernel,
    out_type=jax.ShapeDtypeStruct((NC, N, D), jnp.float32),
    mesh=_mesh,
    scratch_types=[
        [pltpu.VMEM((2, CHUNK), jnp.int32)] * NB,
        [pltpu.VMEM((CHUNK, D), jnp.float32)] * NB,
        [pltpu.SemaphoreType.DMA] * NB,
        pltpu.VMEM_SHARED((N, D), jnp.float32),
    ],
)
def _aggregate_kernel(table_hbm, sd_hbm, out_hbm, sdi, rows, gsem, acc):
    cid = lax.axis_index("c")
    sid = lax.axis_index("s")
    wid = cid * NS + sid

    _blocked_copy(sid, lambda s: table_hbm.at[s], lambda s: acc.at[s])
    plsc.subcore_barrier()

    base = wid * NFULL

    @pl.loop(0, NGRP)
    def _(g):
        c0 = base + g * NB
        for b in range(NB):
            pltpu.sync_copy(sd_hbm.at[c0 + b], sdi[b])
            pltpu.async_copy(table_hbm.at[sdi[b].at[0]], rows[b], gsem[b])
        for b in range(NB):
            pltpu.make_async_copy(table_hbm.at[sdi[b].at[0]], rows[b],
                                  gsem[b]).wait()
            pltpu.sync_copy(rows[b], acc.at[sdi[b].at[1]], add=True)

    @pl.when(wid < NCHT - NW * NFULL)
    def _():
        pltpu.sync_copy(sd_hbm.at[NW * NFULL + wid], sdi[0])
        pltpu.async_copy(table_hbm.at[sdi[0].at[0]], rows[0], gsem[0]).wait()
        pltpu.sync_copy(rows[0], acc.at[sdi[0].at[1]], add=True)

    plsc.subcore_barrier()
    _blocked_copy(sid, lambda s: acc.at[s], lambda s: out_hbm.at[cid, s])


_ROWS = 1000


def _mm_body(x_ref, w_ref, o_ref):
    o_ref[...] = lax.dot_general(
        x_ref[...], w_ref[...], (((1,), (0,)), ((), ())),
        preferred_element_type=jnp.float32,
        precision=lax.Precision.HIGHEST,
    )


def _matmul(x, w):
    return pl.pallas_call(
        _mm_body,
        grid=(N // _ROWS,),
        in_specs=[
            pl.BlockSpec((_ROWS, D), lambda i: (i, 0)),
            pl.BlockSpec((D, D), lambda i: (0, 0)),
        ],
        out_specs=pl.BlockSpec((_ROWS, D), lambda i: (i, 0)),
        out_shape=jax.ShapeDtypeStruct((N, D), jnp.float32),
    )(x, w)


def _dis_block(dp_ref):
    deg = 1.0 + dp_ref[0, :, 0:1] + dp_ref[1, :, 0:1]
    return lax.rsqrt(deg)


def _scale_body(dp_ref, h_ref, o_ref):
    o_ref[...] = h_ref[...] * _dis_block(dp_ref)


def _scale(deg_parts, h):
    return pl.pallas_call(
        _scale_body,
        grid=(N // _ROWS,),
        in_specs=[
            pl.BlockSpec((NC, _ROWS, DEGW), lambda i: (0, i, 0)),
            pl.BlockSpec((_ROWS, D), lambda i: (i, 0)),
        ],
        out_specs=pl.BlockSpec((_ROWS, D), lambda i: (i, 0)),
        out_shape=jax.ShapeDtypeStruct((N, D), jnp.float32),
    )(deg_parts, h)


def _mid_body(p_ref, h_ref, dp_ref, b1_ref, w2_ref, o_ref):
    dis = _dis_block(dp_ref)
    s = p_ref[0] + p_ref[1] - h_ref[...]
    z = jnp.maximum(s * dis + b1_ref[...], 0.0)
    o_ref[...] = lax.dot_general(
        z, w2_ref[...], (((1,), (0,)), ((), ())),
        preferred_element_type=jnp.float32,
        precision=lax.Precision.HIGHEST,
    ) * dis


def _mid(parts, h, deg_parts, b1, w2):
    return pl.pallas_call(
        _mid_body,
        grid=(N // _ROWS,),
        in_specs=[
            pl.BlockSpec((NC, _ROWS, D), lambda i: (0, i, 0)),
            pl.BlockSpec((_ROWS, D), lambda i: (i, 0)),
            pl.BlockSpec((NC, _ROWS, DEGW), lambda i: (0, i, 0)),
            pl.BlockSpec((1, D), lambda i: (0, 0)),
            pl.BlockSpec((D, D), lambda i: (0, 0)),
        ],
        out_specs=pl.BlockSpec((_ROWS, D), lambda i: (i, 0)),
        out_shape=jax.ShapeDtypeStruct((N, D), jnp.float32),
    )(parts, h, deg_parts, b1, w2)


def _final_body(p_ref, h_ref, dp_ref, b2_ref, o_ref):
    dis = _dis_block(dp_ref)
    s = p_ref[0] + p_ref[1] - h_ref[...]
    o_ref[...] = s * dis + b2_ref[...]


def _final(parts, h, deg_parts, b2):
    return pl.pallas_call(
        _final_body,
        grid=(N // _ROWS,),
        in_specs=[
            pl.BlockSpec((NC, _ROWS, D), lambda i: (0, i, 0)),
            pl.BlockSpec((_ROWS, D), lambda i: (i, 0)),
            pl.BlockSpec((NC, _ROWS, DEGW), lambda i: (0, i, 0)),
            pl.BlockSpec((1, D), lambda i: (0, 0)),
        ],
        out_specs=pl.BlockSpec((_ROWS, D), lambda i: (i, 0)),
        out_shape=jax.ShapeDtypeStruct((N, D), jnp.float32),
    )(parts, h, deg_parts, b2)


def kernel(x, edge_index, W1, b1, W2, b2):
    src = edge_index[0].astype(jnp.int32)
    dst = edge_index[1].astype(jnp.int32)
    b1r = b1.reshape(1, D)
    b2r = b2.reshape(1, D)

    sd = jnp.stack([src, dst]).reshape(2, NCHT, CHUNK).transpose(1, 0, 2)

    deg_parts = _degree_kernel(dst)
    h1_raw = _matmul(x, W1)
    h1 = _scale(deg_parts, h1_raw)
    p1 = _aggregate_kernel(h1, sd)
    h2 = _mid(p1, h1, deg_parts, b1r, W2)
    p2 = _aggregate_kernel(h2, sd)
    return _final(p2, h2, deg_parts, b2r)
